# baseline (speedup 1.0000x reference)
import functools
import numpy as np
import jax
import jax.numpy as jnp
from jax import lax
from jax.experimental import pallas as pl
from jax.experimental.pallas import tpu as pltpu
from jax.experimental.pallas import tpu_sc as plsc

NC, NS = 2, 16
NW = NC * NS
HID = 64
F32 = jnp.float32
I32 = jnp.int32

_MESH = functools.partial(
    plsc.VectorSubcoreMesh, core_axis_name="c", subcore_axis_name="s",
    num_cores=NC, num_subcores=NS)


def _sigmoid(x):
  return 1.0 / (1.0 + jnp.exp(-x))


def _silu(x):
  return x / (1.0 + jnp.exp(-x))


def _sc_gather(table, idx, F, K=80):
  E = idx.shape[0]
  n_per_w = E // NW
  nblk = n_per_w // K
  NB = next(n for n in (5, 4, 2, 1) if nblk % n == 0)
  ngrp = nblk // NB
  assert n_per_w % K == 0, (E, K)
  idx3 = idx.reshape(NW, nblk, K)

  def body(table_hbm, idx_hbm, out_hbm, idx_v, buf_v, gsem, ssem):
    cid = lax.axis_index("c")
    sid = lax.axis_index("s")
    wid = sid * NC + cid
    pltpu.sync_copy(idx_hbm.at[wid], idx_v)
    base = wid * n_per_w

    def grp(g, _):
      for b in range(NB):
        j = g * NB + b
        pltpu.async_copy(table_hbm.at[idx_v.at[j]], buf_v.at[b], gsem)
      for b in range(NB):
        pltpu.make_async_copy(table_hbm.at[idx_v.at[0]], buf_v.at[b],
                              gsem).wait()
      for b in range(NB):
        j = g * NB + b
        pltpu.async_copy(buf_v.at[b], out_hbm.at[pl.ds(base + j * K, K)],
                         ssem)
      for b in range(NB):
        pltpu.make_async_copy(buf_v.at[b],
                              out_hbm.at[pl.ds(base, K)], ssem).wait()
      return 0

    lax.fori_loop(0, ngrp, grp, 0)

  fn = pl.kernel(
      body,
      out_type=jax.ShapeDtypeStruct((E, F), F32),
      mesh=_MESH(),
      scratch_types=[
          pltpu.VMEM((nblk, K), I32),
          pltpu.VMEM((NB, K, F), F32),
          pltpu.SemaphoreType.DMA,
          pltpu.SemaphoreType.DMA,
      ],
  )
  return fn(table, idx3)


_BC = 2000


def _seg_bounds(dst, n_dst, n_pad):
  E = dst.shape[0]
  perm = jnp.argsort(dst)
  sd = dst[perm]
  ns = jnp.arange(n_dst, dtype=I32)
  start = jnp.searchsorted(sd, ns, side="left").astype(I32)
  end = jnp.searchsorted(sd, ns, side="right").astype(I32)
  empty = end <= start
  e1 = jnp.where(empty, E, end - 1)
  e0 = jnp.where(empty | (start == 0), E, start - 1)
  pad = jnp.full((n_pad - n_dst,), E, I32)
  e1 = jnp.concatenate([e1, pad])
  e0 = jnp.concatenate([e0, pad])
  return perm.astype(I32), e1, e0


def _csum_blocks(x):
  E = x.shape[0]
  nb = E // _BC

  def body(x_ref, cs_ref, tot_ref):
    s = x_ref[...]
    sh = 1
    while sh < _BC:
      z = jnp.zeros((sh, 128), F32)
      s = s + jnp.concatenate([z, s[:_BC - sh]], axis=0)
      sh *= 2
    cs_ref[...] = s
    tot_ref[...] = s[_BC - 1:_BC].reshape(1, 1, 128)

  return pl.pallas_call(
      body,
      grid=(nb,),
      in_specs=[pl.BlockSpec((_BC, 128), lambda i: (i, 0))],
      out_specs=[pl.BlockSpec((_BC, 128), lambda i: (i, 0)),
                 pl.BlockSpec((1, 1, 128), lambda i: (i, 0, 0))],
      out_shape=[jax.ShapeDtypeStruct((E, 128), F32),
                 jax.ShapeDtypeStruct((nb, 1, 128), F32)],
  )(x)


def _bp_excl(tot):
  nb = tot.shape[0]

  def body(t_ref, o_ref):
    s = t_ref[...]
    sh = 1
    while sh < nb:
      z = jnp.zeros((sh, 128), F32)
      s = s + jnp.concatenate([z, s[:nb - sh]], axis=0)
      sh *= 2
    o_ref[...] = jnp.concatenate(
        [jnp.zeros((1, 128), F32), s[:nb - 1]], axis=0)

  return pl.pallas_call(
      body,
      grid=(1,),
      in_specs=[pl.BlockSpec((nb, 128), lambda i: (0, 0))],
      out_specs=pl.BlockSpec((nb, 128), lambda i: (0, 0)),
      out_shape=jax.ShapeDtypeStruct((nb, 128), F32),
  )(tot)


def _segment_sums(payload, perm, e1, e0, n_dst):
  E = payload.shape[0]
  sorted_pay = _sc_gather(payload, perm, 128)
  cs, tot = _csum_blocks(sorted_pay)
  bp = _bp_excl(tot.reshape(-1, 128))
  cs_ext = jnp.concatenate([cs, jnp.zeros((1, 128), F32)], axis=0)
  bp_ext = jnp.concatenate([bp, jnp.zeros((1, 128), F32)], axis=0)
  g1 = _sc_gather(cs_ext, e1, 128)[:n_dst]
  g0 = _sc_gather(cs_ext, e0, 128)[:n_dst]
  b1 = _sc_gather(bp_ext, e1 // _BC, 128)[:n_dst]
  b0 = _sc_gather(bp_ext, e0 // _BC, 128)[:n_dst]
  return g1, g0, b1, b0


_BE = 2000
_BT = 1280


def _row_grid(n):
  return n // _BE


def _eggc_tables(x, p):
  n = x.shape[0]

  def body(x_ref, wa_ref, ba_ref, wb_ref, bb_ref, wh_ref, bh_ref,
           wc_ref, bc_ref, s_ref, d_ref, c_ref):
    xb = x_ref[...]
    a = jnp.dot(xb, wa_ref[...], preferred_element_type=F32) + ba_ref[...]
    b = jnp.dot(xb, wb_ref[...], preferred_element_type=F32) + bb_ref[...]
    h = jnp.dot(xb, wh_ref[...], preferred_element_type=F32) + bh_ref[...]
    c = jnp.dot(xb, wc_ref[...], preferred_element_type=F32) + bc_ref[...]
    s_ref[...] = jnp.concatenate([a, h], axis=1)
    d_ref[...] = jnp.concatenate([b, jnp.zeros_like(b)], axis=1)
    c_ref[...] = c

  wspec = pl.BlockSpec((HID, HID), lambda i: (0, 0))
  bspec = pl.BlockSpec((1, HID), lambda i: (0, 0))
  xspec = pl.BlockSpec((_BE, HID), lambda i: (i, 0))
  wide = pl.BlockSpec((_BE, 2 * HID), lambda i: (i, 0))
  args = []
  for k in ("src_gate", "dst_gate", "dst_update", "src_update"):
    args += [p[k]["w"].T, p[k]["b"].reshape(1, HID)]
  return pl.pallas_call(
      body,
      grid=(_row_grid(n),),
      in_specs=[xspec] + [wspec, bspec] * 4,
      out_specs=[wide, wide, xspec],
      out_shape=[jax.ShapeDtypeStruct((n, 2 * HID), F32),
                 jax.ShapeDtypeStruct((n, 2 * HID), F32),
                 jax.ShapeDtypeStruct((n, HID), F32)],
  )(x, *args)


def _edge_stage_a(gs, gd, y, w_edge, b_edge):
  E = y.shape[0]

  def body(gs_ref, gd_ref, y_ref, w_ref, b_ref, m_ref, pay_ref, st_ref):
    gsb = gs_ref[...]
    m = gsb[:, :HID] + gd_ref[...][:, :HID] + jnp.dot(
        y_ref[...], w_ref[...], preferred_element_type=F32) + b_ref[...]
    sig = _sigmoid(m)
    m_ref[...] = m
    pay_ref[...] = jnp.concatenate([sig, sig * gsb[:, HID:]], axis=1)

    @pl.when(pl.program_id(0) == 0)
    def _():
      st_ref[...] = jnp.zeros_like(st_ref)

    st_ref[...] += jnp.concatenate(
        [jnp.sum(m, 0, keepdims=True), jnp.sum(m * m, 0, keepdims=True)], 0)

  rspec = pl.BlockSpec((_BE, HID), lambda i: (i, 0))
  wide = pl.BlockSpec((_BE, 2 * HID), lambda i: (i, 0))
  return pl.pallas_call(
      body,
      grid=(_row_grid(E),),
      in_specs=[wide, wide, rspec,
                pl.BlockSpec((HID, HID), lambda i: (0, 0)),
                pl.BlockSpec((1, HID), lambda i: (0, 0))],
      out_specs=[rspec, wide,
                 pl.BlockSpec((2, HID), lambda i: (0, 0))],
      out_shape=[jax.ShapeDtypeStruct((E, HID), F32),
                 jax.ShapeDtypeStruct((E, 2 * HID), F32),
                 jax.ShapeDtypeStruct((2, HID), F32)],
  )(gs, gd, y, w_edge.T, b_edge.reshape(1, HID))


def _node_stage_t(cx, g1, g0, b1, b0):
  n = cx.shape[0]

  def body(cx_ref, g1_ref, g0_ref, b1_ref, b0_ref, t_ref, st_ref):
    s = (g1_ref[...] - g0_ref[...]) + (b1_ref[...] - b0_ref[...])
    t = cx_ref[...] + s[:, HID:] / (s[:, :HID] + 1e-6)
    t_ref[...] = t

    @pl.when(pl.program_id(0) == 0)
    def _():
      st_ref[...] = jnp.zeros_like(st_ref)

    st_ref[...] += jnp.concatenate(
        [jnp.sum(t, 0, keepdims=True), jnp.sum(t * t, 0, keepdims=True)], 0)

  wide = pl.BlockSpec((_BE, 2 * HID), lambda i: (i, 0))
  return pl.pallas_call(
      body,
      grid=(_row_grid(n),),
      in_specs=[pl.BlockSpec((_BE, HID), lambda i: (i, 0)),
                wide, wide, wide, wide],
      out_specs=[pl.BlockSpec((_BE, HID), lambda i: (i, 0)),
                 pl.BlockSpec((2, HID), lambda i: (0, 0))],
      out_shape=[jax.ShapeDtypeStruct((n, HID), F32),
                 jax.ShapeDtypeStruct((2, HID), F32)],
  )(cx, g1, g0, b1, b0)


def _residual_bn_silu(x, t, stats, g, b, n_rows):
  n = x.shape[0]

  def body(x_ref, t_ref, st_ref, g_ref, b_ref, o_ref):
    mean = st_ref[0:1] * (1.0 / n_rows)
    var = st_ref[1:2] * (1.0 / n_rows) - mean * mean
    rstd = 1.0 / jnp.sqrt(var + 1e-5)
    h = (t_ref[...] - mean) * rstd * g_ref[...] + b_ref[...]
    o_ref[...] = x_ref[...] + _silu(h)

  rspec = pl.BlockSpec((_BE, HID), lambda i: (i, 0))
  cspec = pl.BlockSpec((1, HID), lambda i: (0, 0))
  return pl.pallas_call(
      body,
      grid=(_row_grid(n),),
      in_specs=[rspec, rspec, pl.BlockSpec((2, HID), lambda i: (0, 0)),
                cspec, cspec],
      out_specs=rspec,
      out_shape=jax.ShapeDtypeStruct((n, HID), F32),
  )(x, t, stats, g.reshape(1, HID), b.reshape(1, HID))


def _rbf_lin_t(inp, w1, b1, bins, vmin, vmax, from_r):
  E = inp.shape[-1]
  gamma = (bins - 1.0) / (vmax - vmin)
  gamma = gamma * gamma
  step = (vmax - vmin) / (bins - 1.0)

  def body(in_ref, w_ref, b_ref, o_ref, st_ref):
    if from_r:
      rb = in_ref[...]
      v = jnp.sqrt(rb[0:1] * rb[0:1] + rb[1:2] * rb[1:2] +
                   rb[2:3] * rb[2:3])
    else:
      v = in_ref[...]
    cent = vmin + step * lax.broadcasted_iota(I32, (bins, 1), 0).astype(F32)
    rbf = jnp.exp(-gamma * (v - cent) * (v - cent))
    u = jnp.dot(w_ref[...], rbf, preferred_element_type=F32) + b_ref[...]
    o_ref[...] = u

    @pl.when(pl.program_id(0) == 0)
    def _():
      st_ref[...] = jnp.zeros_like(st_ref)

    st_ref[...] += jnp.concatenate(
        [jnp.sum(u, 1, keepdims=True), jnp.sum(u * u, 1, keepdims=True)], 1)

  in_spec = (pl.BlockSpec((3, _BT), lambda i: (0, i)) if from_r
             else pl.BlockSpec((1, _BT), lambda i: (0, i)))
  return pl.pallas_call(
      body,
      grid=(E // _BT,),
      in_specs=[in_spec,
                pl.BlockSpec((HID, bins), lambda i: (0, 0)),
                pl.BlockSpec((HID, 1), lambda i: (0, 0))],
      out_specs=[pl.BlockSpec((HID, _BT), lambda i: (0, i)),
                 pl.BlockSpec((HID, 2), lambda i: (0, 0))],
      out_shape=[jax.ShapeDtypeStruct((HID, E), F32),
                 jax.ShapeDtypeStruct((HID, 2), F32)],
  )(inp, w1, b1.reshape(HID, 1))


def _bn_silu_lin_t(ut, stats, g, b, w2, b2, n_rows):
  E = ut.shape[1]

  def body(u_ref, st_ref, g_ref, b_ref, w_ref, b2_ref, o_ref, st2_ref):
    mean = st_ref[:, 0:1] * (1.0 / n_rows)
    var = st_ref[:, 1:2] * (1.0 / n_rows) - mean * mean
    rstd = 1.0 / jnp.sqrt(var + 1e-5)
    a = _silu((u_ref[...] - mean) * rstd * g_ref[...] + b_ref[...])
    u2 = jnp.dot(w_ref[...], a, preferred_element_type=F32) + b2_ref[...]
    o_ref[...] = u2

    @pl.when(pl.program_id(0) == 0)
    def _():
      st2_ref[...] = jnp.zeros_like(st2_ref)

    st2_ref[...] += jnp.concatenate(
        [jnp.sum(u2, 1, keepdims=True), jnp.sum(u2 * u2, 1, keepdims=True)],
        1)

  tspec = pl.BlockSpec((HID, _BT), lambda i: (0, i))
  cspec = pl.BlockSpec((HID, 1), lambda i: (0, 0))
  sspec = pl.BlockSpec((HID, 2), lambda i: (0, 0))
  return pl.pallas_call(
      body,
      grid=(E // _BT,),
      in_specs=[tspec, sspec, cspec, cspec,
                pl.BlockSpec((HID, HID), lambda i: (0, 0)), cspec],
      out_specs=[tspec, sspec],
      out_shape=[jax.ShapeDtypeStruct((HID, E), F32),
                 jax.ShapeDtypeStruct((HID, 2), F32)],
  )(ut, stats, g.reshape(HID, 1), b.reshape(HID, 1), w2,
    b2.reshape(HID, 1))


def _bn_silu_transpose(ut, stats, g, b, n_rows):
  E = ut.shape[1]

  def body(u_ref, st_ref, g_ref, b_ref, o_ref):
    mean = st_ref[:, 0:1] * (1.0 / n_rows)
    var = st_ref[:, 1:2] * (1.0 / n_rows) - mean * mean
    rstd = 1.0 / jnp.sqrt(var + 1e-5)
    a = _silu((u_ref[...] - mean) * rstd * g_ref[...] + b_ref[...])
    o_ref[...] = jnp.transpose(a)

  cspec = pl.BlockSpec((HID, 1), lambda i: (0, 0))
  return pl.pallas_call(
      body,
      grid=(E // _BT,),
      in_specs=[pl.BlockSpec((HID, _BT), lambda i: (0, i)),
                pl.BlockSpec((HID, 2), lambda i: (0, 0)), cspec, cspec],
      out_specs=pl.BlockSpec((_BT, HID), lambda i: (i, 0)),
      out_shape=jax.ShapeDtypeStruct((E, HID), F32),
  )(ut, stats, g.reshape(HID, 1), b.reshape(HID, 1))


def _inter_table(x, w_src, b_src, w_dst, b_dst):
  n = x.shape[0]

  def body(x_ref, ws_ref, bs_ref, wd_ref, bd_ref, o_ref):
    xb = x_ref[...]
    es = jnp.dot(xb, ws_ref[...], preferred_element_type=F32) + bs_ref[...]
    ed = jnp.dot(xb, wd_ref[...], preferred_element_type=F32) + bd_ref[...]
    o_ref[...] = jnp.concatenate(
        [es, ed, jnp.zeros((xb.shape[0], 120), F32)], axis=1)

  return pl.pallas_call(
      body,
      grid=(_row_grid(n),),
      in_specs=[pl.BlockSpec((_BE, HID), lambda i: (i, 0)),
                pl.BlockSpec((HID, 4), lambda i: (0, 0)),
                pl.BlockSpec((1, 4), lambda i: (0, 0)),
                pl.BlockSpec((HID, 4), lambda i: (0, 0)),
                pl.BlockSpec((1, 4), lambda i: (0, 0))],
      out_specs=pl.BlockSpec((_BE, 128), lambda i: (i, 0)),
      out_shape=jax.ShapeDtypeStruct((n, 128), F32),
  )(x, w_src.T, b_src.reshape(1, 4), w_dst.T, b_dst.reshape(1, 4))


def _potential_energy(r, gs, gd, y, w_fc, b_fc):
  E = y.shape[0]
  Dc = 0.5 * (4.0 - 3.8)
  Rc = 4.0 - Dc

  def body(r_ref, gs_ref, gd_ref, y_ref, w_ref, b_ref, e_ref):
    rb = r_ref[...]
    bl = jnp.sqrt(jnp.sum(rb * rb, axis=1, keepdims=True))
    pair = jnp.exp(gs_ref[...][:, 0:4] + gd_ref[...][:, 4:8])
    bo = _sigmoid(jnp.dot(y_ref[...], w_ref[...],
                          preferred_element_type=F32) + b_ref[...])
    f_rep = pair[:, 0:1] * jnp.exp(-pair[:, 1:2] * bl)
    f_att = pair[:, 2:3] * jnp.exp(-pair[:, 3:4] * bl)
    c = jnp.where(bl < Rc - Dc, jnp.ones_like(bl),
                  0.5 - 0.5 * jnp.sin(np.pi * (bl - Rc) / (2 * Dc)))
    c = jnp.where(bl > Rc + Dc, jnp.zeros_like(bl), c)
    v = c * (f_rep - bo * f_att)

    @pl.when(pl.program_id(0) == 0)
    def _():
      e_ref[...] = jnp.zeros_like(e_ref)

    e_ref[...] += jnp.sum(v).reshape(1, 1)

  return pl.pallas_call(
      body,
      grid=(_row_grid(E),),
      in_specs=[pl.BlockSpec((_BE, 3), lambda i: (i, 0)),
                pl.BlockSpec((_BE, 128), lambda i: (i, 0)),
                pl.BlockSpec((_BE, 128), lambda i: (i, 0)),
                pl.BlockSpec((_BE, HID), lambda i: (i, 0)),
                pl.BlockSpec((HID, 1), lambda i: (0, 0)),
                pl.BlockSpec((1, 1), lambda i: (0, 0))],
      out_specs=pl.BlockSpec((1, 1), lambda i: (0, 0)),
      out_shape=jax.ShapeDtypeStruct((1, 1), F32),
  )(r, gs, gd, y, w_fc.reshape(HID, 1), b_fc.reshape(1, 1))


def _eggc(p, src, dst, x, y, n_nodes, seg):
  perm, e1, e0 = seg
  src_tab, dst_tab, cx = _eggc_tables(x, p)
  gs = _sc_gather(src_tab, src, 2 * HID)
  gd = _sc_gather(dst_tab, dst, 2 * HID)
  m, payload, st_m = _edge_stage_a(
      gs, gd, y, p["edge_gate"]["w"], p["edge_gate"]["b"])
  g1, g0, b1, b0 = _segment_sums(payload, perm, e1, e0, n_nodes)
  t, st_t = _node_stage_t(cx, g1, g0, b1, b0)
  x_new = _residual_bn_silu(x, t, st_t, p["bn_nodes"]["g"],
                            p["bn_nodes"]["b"], n_nodes)
  y_new = _residual_bn_silu(y, m, st_m, p["bn_edges"]["g"],
                            p["bn_edges"]["b"], y.shape[0])
  return x_new, y_new


def kernel(atom_numbers, edge_index, r, lg_edge_index, angle_h, params):
  n_nodes = atom_numbers.shape[0]
  n_edges = r.shape[0]
  src = edge_index[0]
  dst = edge_index[1]
  lsrc = lg_edge_index[0]
  ldst = lg_edge_index[1]

  pad_n = ((n_nodes + NW * 80 - 1) // (NW * 80)) * (NW * 80)
  a_idx = jnp.concatenate(
      [atom_numbers.astype(I32),
       jnp.zeros((pad_n - n_nodes,), I32)])
  emb128 = jnp.pad(params["emb"], ((0, 0), (0, 128 - HID)))
  x = _sc_gather(emb128, a_idx, 128)[:n_nodes, :HID]
  x_initial = x

  rt = r.T
  u1t, st1 = _rbf_lin_t(rt, params["edge_mlp1"]["lin"]["w"],
                        params["edge_mlp1"]["lin"]["b"], 80, 0.0, 8.0, True)
  u2t, st2 = _bn_silu_lin_t(u1t, st1, params["edge_mlp1"]["bn"]["g"],
                            params["edge_mlp1"]["bn"]["b"],
                            params["edge_mlp2"]["lin"]["w"],
                            params["edge_mlp2"]["lin"]["b"], n_edges)
  y = _bn_silu_transpose(u2t, st2, params["edge_mlp2"]["bn"]["g"],
                         params["edge_mlp2"]["bn"]["b"], n_edges)

  n_lg = angle_h.shape[0]
  v1t, sa1 = _rbf_lin_t(angle_h.reshape(1, -1),
                        params["angle_mlp1"]["lin"]["w"],
                        params["angle_mlp1"]["lin"]["b"], 40, -1.0, 1.0,
                        False)
  v2t, sa2 = _bn_silu_lin_t(v1t, sa1, params["angle_mlp1"]["bn"]["g"],
                            params["angle_mlp1"]["bn"]["b"],
                            params["angle_mlp2"]["lin"]["w"],
                            params["angle_mlp2"]["lin"]["b"], n_lg)
  z = _bn_silu_transpose(v2t, sa2, params["angle_mlp2"]["bn"]["g"],
                         params["angle_mlp2"]["bn"]["b"], n_lg)

  seg_g = _seg_bounds(dst, n_nodes, 12800)
  seg_l = _seg_bounds(ldst, n_edges, n_edges)
  for lp in params["alignn"]:
    x, m = _eggc(lp["node"], src, dst, x, y, n_nodes, seg_g)
    y, z = _eggc(lp["edge"], lsrc, ldst, m, z, n_edges, seg_l)
  for gp in params["gcn"]:
    x, y = _eggc(gp, src, dst, x, y, n_nodes, seg_g)

  table = _inter_table(x_initial, params["inter_src"]["w"],
                       params["inter_src"]["b"], params["inter_dst"]["w"],
                       params["inter_dst"]["b"])
  gs = _sc_gather(table, src, 128)
  gd = _sc_gather(table, dst, 128)
  energy = _potential_energy(r, gs, gd, y, params["fc"]["w"],
                             params["fc"]["b"])
  return jnp.squeeze(energy)
